# Initial kernel scaffold; baseline (speedup 1.0000x reference)
import jax
import jax.numpy as jnp
from jax.experimental import pallas as pl


def kernel(efeat, initial_efeats, edge_index, W_m, b_m, W_a, b_a):
    raise NotImplementedError("write your pallas kernel here")


# speedup vs baseline: 24.0039x; 24.0039x over previous
import functools

import jax
import jax.numpy as jnp
from jax import lax
from jax.experimental import pallas as pl
from jax.experimental.pallas import tpu as pltpu
from jax.experimental.pallas import tpu_sc as plsc

F = 16
FF = F * F
E = 100000
N_NODES = 10000
N_PAD = 10240

TC_BLK = 1000
TC_GRID = E // TC_BLK

SC_B = 160
SC_NBATCH = E // SC_B
SC_ROWS_PER_SUB = N_PAD // 16
IDXW = 80
NIDX = SC_B // IDXW


def _tc_body(x_ref, ih_ref, wm_ref, bm_ref, wa_ref, ba_ref, k_ref,
             se0_ref, se1_ref, sh0_ref, sh1_ref, d0_ref, d1_ref):
    x = x_ref[:]
    ih = ih_ref[:]
    hp = jax.lax.Precision.HIGHEST
    m = jnp.dot(x, wm_ref[:],
                preferred_element_type=jnp.float32) + bm_ref[:]
    a = jnp.dot(x, wa_ref[:],
                preferred_element_type=jnp.float32) + ba_ref[:]
    hrep = jnp.dot(x, k_ref[:], precision=hp,
                   preferred_element_type=jnp.float32)
    ihrep = jnp.dot(ih, k_ref[:], precision=hp,
                    preferred_element_type=jnp.float32)
    exp_e = jnp.exp(a * hrep)
    h1 = exp_e * (m * hrep)
    exp_ie = jnp.exp(a * ihrep)
    ih1 = exp_ie * (m * ihrep)
    de = exp_ie - exp_e
    dm = ih1 - h1
    se0_ref[:] = exp_e[:, :128]
    se1_ref[:] = exp_e[:, 128:]
    sh0_ref[:] = h1[:, :128]
    sh1_ref[:] = h1[:, 128:]
    d0_ref[:, :128] = de[:, :128]
    d0_ref[:, 128:] = dm[:, :128]
    d1_ref[:, :128] = de[:, 128:]
    d1_ref[:, 128:] = dm[:, 128:]


def _tc_phase(efeat, initial_efeats, W_m, b_m, W_a, b_a, kmat):
    eb = pl.BlockSpec((TC_BLK, F), lambda i: (i, 0))
    wb = pl.BlockSpec((F, FF), lambda i: (0, 0))
    bb = pl.BlockSpec((1, FF), lambda i: (0, 0))
    o128 = pl.BlockSpec((TC_BLK, 128), lambda i: (i, 0))
    o256 = pl.BlockSpec((TC_BLK, 256), lambda i: (i, 0))
    f32 = jnp.float32
    return pl.pallas_call(
        _tc_body,
        grid=(TC_GRID,),
        in_specs=[eb, eb, wb, bb, wb, bb, wb],
        out_specs=[o128] * 4 + [o256] * 2,
        out_shape=[jax.ShapeDtypeStruct((E, 128), f32)] * 4
                  + [jax.ShapeDtypeStruct((E, 256), f32)] * 2,
    )(efeat, initial_efeats, W_m, b_m.reshape(1, FF), W_a,
      b_a.reshape(1, FF), kmat)


def _sc_mesh():
    return plsc.VectorSubcoreMesh(core_axis_name="c", subcore_axis_name="s")


def _scatter_kernel(se0, se1, sh0, sh1, dst, zrows,
                    a0, a1, a2, a3, vals, idx0, idx1, acc):
    s = lax.axis_index("s")
    c = lax.axis_index("c")
    idxs = (idx0, idx1)

    def do_pass(src_hbm, out_hbm):
        r0 = s * SC_ROWS_PER_SUB
        pltpu.sync_copy(zrows.at[pl.ds(r0, SC_ROWS_PER_SUB)],
                        acc.at[pl.ds(r0, SC_ROWS_PER_SUB)])
        plsc.subcore_barrier()
        for k in range(40):
            b = k * 16 + s

            @pl.when(b < SC_NBATCH)
            def _():
                pltpu.sync_copy(src_hbm.at[pl.ds(b * SC_B, SC_B)], vals)
                for r in range(NIDX):
                    pltpu.sync_copy(
                        dst.at[pl.ds(b * SC_B + r * IDXW, IDXW)], idxs[r])
                for r in range(NIDX):
                    pltpu.sync_copy(vals.at[pl.ds(r * IDXW, IDXW)],
                                    acc.at[idxs[r]], add=True)
        plsc.subcore_barrier()
        pltpu.sync_copy(acc.at[pl.ds(r0, SC_ROWS_PER_SUB)],
                        out_hbm.at[pl.ds(r0, SC_ROWS_PER_SUB)])
        plsc.subcore_barrier()

    @pl.when(c == 0)
    def _():
        do_pass(se0, a0)
        do_pass(se1, a1)

    @pl.when(c == 1)
    def _():
        do_pass(sh0, a2)
        do_pass(sh1, a3)


def _sc_scatter(se0, se1, sh0, sh1, dst, zrows):
    f32 = jnp.float32
    return pl.kernel(
        _scatter_kernel,
        mesh=_sc_mesh(),
        out_type=[jax.ShapeDtypeStruct((N_PAD, 128), f32)] * 4,
        scratch_types=[
            pltpu.VMEM((SC_B, 128), f32),
            pltpu.VMEM((IDXW,), jnp.int32),
            pltpu.VMEM((IDXW,), jnp.int32),
            pltpu.VMEM_SHARED((N_PAD, 128), f32),
        ],
    )(se0, se1, sh0, sh1, dst, zrows)


def _gather_kernel(a0, a1, a2, a3, d0, d1, src, out,
                   g_e, g_m, dbuf, obuf, idx0, idx1):
    s = lax.axis_index("s")
    c = lax.axis_index("c")
    wid = s * 2 + c
    idxs = (idx0, idx1)

    for k in range(20):
        b = k * 32 + wid

        @pl.when(b < SC_NBATCH)
        def _():
            e0 = b * SC_B
            for r in range(NIDX):
                pltpu.sync_copy(src.at[pl.ds(e0 + r * IDXW, IDXW)], idxs[r])
            for h in range(2):
                ae = (a0, a1)[h]
                am = (a2, a3)[h]
                dh = (d0, d1)[h]
                pltpu.sync_copy(dh.at[pl.ds(e0, SC_B)], dbuf)
                for r in range(NIDX):
                    pltpu.sync_copy(ae.at[idxs[r]],
                                    g_e.at[pl.ds(r * IDXW, IDXW)])
                    pltpu.sync_copy(am.at[idxs[r]],
                                    g_m.at[pl.ds(r * IDXW, IDXW)])

                def body(e, _, h=h):
                    acc = jnp.zeros((F,), jnp.float32)
                    for i in range(8):
                        num = g_m[e, pl.ds(i * F, F)] + \
                            dbuf[e, pl.ds(128 + i * F, F)]
                        den = g_e[e, pl.ds(i * F, F)] + \
                            dbuf[e, pl.ds(i * F, F)]
                        acc = acc + num / den
                    if h == 0:
                        obuf[e, :] = acc
                    else:
                        obuf[e, :] = obuf[e, :] + acc
                    return 0

                lax.fori_loop(0, SC_B, body, 0)
            pltpu.sync_copy(obuf, out.at[pl.ds(e0, SC_B)])


def _sc_gather(a0, a1, a2, a3, d0, d1, src):
    f32 = jnp.float32
    return pl.kernel(
        _gather_kernel,
        mesh=_sc_mesh(),
        out_type=jax.ShapeDtypeStruct((E, F), f32),
        scratch_types=[
            pltpu.VMEM((SC_B, 128), f32),
            pltpu.VMEM((SC_B, 128), f32),
            pltpu.VMEM((SC_B, 256), f32),
            pltpu.VMEM((SC_B, F), f32),
            pltpu.VMEM((IDXW,), jnp.int32),
            pltpu.VMEM((IDXW,), jnp.int32),
        ],
    )(a0, a1, a2, a3, d0, d1, src)


def kernel(efeat, initial_efeats, edge_index, W_m, b_m, W_a, b_a):
    kmat = jnp.kron(jnp.eye(F, dtype=jnp.float32),
                    jnp.ones((1, F), jnp.float32))
    se0, se1, sh0, sh1, d0, d1 = _tc_phase(
        efeat, initial_efeats, W_m, b_m, W_a, b_a, kmat)
    zrows = jnp.zeros((N_PAD, 128), jnp.float32)
    a0, a1, a2, a3 = _sc_scatter(se0, se1, sh0, sh1, edge_index[1], zrows)
    return _sc_gather(a0, a1, a2, a3, d0, d1, edge_index[0])
